# Initial kernel scaffold; baseline (speedup 1.0000x reference)
import functools

import jax
import jax.numpy as jnp
from jax import lax
from jax.experimental import pallas as pl
from jax.experimental.pallas import tpu as pltpu
from jax.experimental.pallas import tpu_sc as plsc

NC = 2
NS = 16
NW = NC * NS
LANES = 16
K = 80


def _mesh():
  return plsc.VectorSubcoreMesh(core_axis_name="c", subcore_axis_name="s",
                                num_cores=NC)


@functools.partial(jax.jit, static_argnames=("n", "r"))
def _sc_prep(src, dst, typ, zeros_hist, *, n, r):
  e = src.shape[0]
  nr = n * r
  per_tile_a = e // NS
  per_tile_b = e // NW
  nb_a = per_tile_a // K
  nb_b = per_tile_b // K
  stripe = nr // NS

  @functools.partial(
      pl.kernel,
      out_type=(jax.ShapeDtypeStruct((e,), jnp.int32),
                jax.ShapeDtypeStruct((e,), jnp.float32)),
      mesh=_mesh(),
      scratch_types=[
          pltpu.VMEM((K,), jnp.int32),
          pltpu.VMEM((K,), jnp.int32),
          pltpu.VMEM((K,), jnp.int32),
          pltpu.VMEM((K,), jnp.int32),
          pltpu.VMEM((K,), jnp.float32),
          pltpu.VMEM((K,), jnp.int32),
          pltpu.VMEM((K,), jnp.float32),
          pltpu.VMEM((nr,), jnp.float32),
          pltpu.VMEM_SHARED((nr,), jnp.float32),
      ],
  )
  def prep(src_h, dst_h, typ_h, zeros_h, g_h, w_h,
           src_v, dst_v, typ_v, key_v, ones_v, g_v, w_v, hist_v, hist_sh):
    sid = lax.axis_index("s")
    cid = lax.axis_index("c")
    wid = sid * NC + cid

    pltpu.sync_copy(zeros_h.at[pl.ds(sid * stripe, stripe)],
                    hist_sh.at[pl.ds(sid * stripe, stripe)])
    for j in range(K // LANES):
      ones_v[pl.ds(j * LANES, LANES)] = jnp.full((LANES,), 1.0, jnp.float32)
    plsc.subcore_barrier()

    base_a = sid * per_tile_a

    def body_a(b, carry):
      off = base_a + b * K
      pltpu.sync_copy(dst_h.at[pl.ds(off, K)], dst_v)
      pltpu.sync_copy(typ_h.at[pl.ds(off, K)], typ_v)
      for j in range(K // LANES):
        sl = pl.ds(j * LANES, LANES)
        key_v[sl] = dst_v[sl] * r + typ_v[sl]
      pltpu.sync_copy(ones_v, hist_sh.at[key_v], add=True)
      return carry

    lax.fori_loop(0, nb_a, body_a, 0)
    plsc.subcore_barrier()

    pltpu.sync_copy(hist_sh, hist_v)
    base_b = wid * per_tile_b

    def body_b(b, carry):
      off = base_b + b * K
      pltpu.sync_copy(src_h.at[pl.ds(off, K)], src_v)
      pltpu.sync_copy(dst_h.at[pl.ds(off, K)], dst_v)
      pltpu.sync_copy(typ_h.at[pl.ds(off, K)], typ_v)
      for j in range(K // LANES):
        sl = pl.ds(j * LANES, LANES)
        t = typ_v[sl]
        g_v[sl] = src_v[sl] * (r + 1) + t
        cnt = plsc.load_gather(hist_v, [dst_v[sl] * r + t])
        w_v[sl] = 1.0 / jnp.maximum(cnt, 1.0)
      pltpu.sync_copy(g_v, g_h.at[pl.ds(off, K)])
      pltpu.sync_copy(w_v, w_h.at[pl.ds(off, K)])
      return carry

    lax.fori_loop(0, nb_b, body_b, 0)

  return prep(src, dst, typ, zeros_hist)


@functools.partial(jax.jit, static_argnames=("n", "d"))
def _sc_scatter(table, g, dst, w, zeros_acc, *, n, d):
  e = g.shape[0]
  per_tile = e // NW
  nb = per_tile // K
  stripe = n // NS

  @functools.partial(
      pl.kernel,
      out_type=jax.ShapeDtypeStruct((NC, n, d), jnp.float32),
      mesh=_mesh(),
      scratch_types=[
          pltpu.VMEM((K,), jnp.int32),
          pltpu.VMEM((K,), jnp.int32),
          pltpu.VMEM((K,), jnp.float32),
          pltpu.VMEM((K, d), jnp.float32),
          pltpu.VMEM_SHARED((n, d), jnp.float32),
          pltpu.SemaphoreType.DMA,
      ],
  )
  def scat(t_h, g_h, dst_h, w_h, zeros_h, out_h,
           g_v, dst_v, w_v, rows_v, acc_sh, sem):
    sid = lax.axis_index("s")
    cid = lax.axis_index("c")
    wid = sid * NC + cid

    pltpu.sync_copy(zeros_h.at[pl.ds(sid * stripe, stripe)],
                    acc_sh.at[pl.ds(sid * stripe, stripe)])
    plsc.subcore_barrier()

    base = wid * per_tile

    def body(b, carry):
      off = base + b * K
      pltpu.sync_copy(g_h.at[pl.ds(off, K)], g_v)
      pltpu.sync_copy(dst_h.at[pl.ds(off, K)], dst_v)
      pltpu.sync_copy(w_h.at[pl.ds(off, K)], w_v)
      pltpu.async_copy(t_h.at[g_v], rows_v, sem).wait()

      def scale(i, c2):
        ws = w_v[i]
        for k in range(d // LANES):
          sl = pl.ds(k * LANES, LANES)
          rows_v[i, sl] = rows_v[i, sl] * ws
        return c2

      lax.fori_loop(0, K, scale, 0)
      pltpu.sync_copy(rows_v, acc_sh.at[dst_v], add=True)
      return carry

    lax.fori_loop(0, nb, body, 0)
    plsc.subcore_barrier()
    pltpu.sync_copy(acc_sh.at[pl.ds(sid * stripe, stripe)],
                    out_h.at[cid, pl.ds(sid * stripe, stripe)])

  return scat(table, g, dst, w, zeros_acc)


_BN = 1000


def _tc_first_body(x_ref, w_ref, b_ref, wb_ref, o_ref):
  h = jnp.dot(x_ref[...], w_ref[...], preferred_element_type=jnp.float32)
  h = jnp.maximum(h + b_ref[...], 0.0)
  o_ref[...] = jnp.dot(h, wb_ref[...], preferred_element_type=jnp.float32)


def _tc_first(x, w_in, b_in, wbig):
  n, d = x.shape
  do = wbig.shape[1]
  return pl.pallas_call(
      _tc_first_body,
      grid=(n // _BN,),
      in_specs=[
          pl.BlockSpec((_BN, d), lambda i: (i, 0)),
          pl.BlockSpec((d, d), lambda i: (0, 0)),
          pl.BlockSpec((1, d), lambda i: (0, 0)),
          pl.BlockSpec((d, do), lambda i: (0, 0)),
      ],
      out_specs=pl.BlockSpec((_BN, do), lambda i: (i, 0)),
      out_shape=jax.ShapeDtypeStruct((n, do), jnp.float32),
  )(x, w_in, b_in, wbig)


def _tc_mid_body(p_ref, root_ref, b_ref, wb_ref, o_ref):
  p = p_ref[...]
  h = jnp.maximum(p[0] + p[1] + root_ref[...] + b_ref[...], 0.0)
  o_ref[...] = jnp.dot(h, wb_ref[...], preferred_element_type=jnp.float32)


def _tc_mid(p, t_prev, b, wbig, r):
  n, d = p.shape[1], p.shape[2]
  do = wbig.shape[1]
  return pl.pallas_call(
      _tc_mid_body,
      grid=(n // _BN,),
      in_specs=[
          pl.BlockSpec((2, _BN, d), lambda i: (0, i, 0)),
          pl.BlockSpec((_BN, d), lambda i: (i, r)),
          pl.BlockSpec((1, d), lambda i: (0, 0)),
          pl.BlockSpec((d, do), lambda i: (0, 0)),
      ],
      out_specs=pl.BlockSpec((_BN, do), lambda i: (i, 0)),
      out_shape=jax.ShapeDtypeStruct((n, do), jnp.float32),
  )(p, t_prev, b, wbig)


def _tc_last_body(p_ref, root_ref, b_ref, o_ref):
  p = p_ref[...]
  o_ref[...] = jnp.maximum(p[0] + p[1] + root_ref[...] + b_ref[...], 0.0)


def _tc_last(p, t_prev, b, r):
  n, d = p.shape[1], p.shape[2]
  return pl.pallas_call(
      _tc_last_body,
      grid=(n // _BN,),
      in_specs=[
          pl.BlockSpec((2, _BN, d), lambda i: (0, i, 0)),
          pl.BlockSpec((_BN, d), lambda i: (i, r)),
          pl.BlockSpec((1, d), lambda i: (0, 0)),
      ],
      out_specs=pl.BlockSpec((_BN, d), lambda i: (i, 0)),
      out_shape=jax.ShapeDtypeStruct((n, d), jnp.float32),
  )(p, t_prev, b)


def kernel(x, edge_index, edge_attr, W_in, b_in, rel_W, root_W, root_b):
  n, d = x.shape
  e = edge_index.shape[1]
  nl, r = rel_W.shape[0], rel_W.shape[1]

  src = edge_index[0]
  dst = edge_index[1]
  typ = edge_attr[:, 1].astype(jnp.int32)
  edge_distance = edge_attr[:, 0].astype(jnp.float32)

  zeros_hist = jnp.zeros((n * r,), jnp.float32)
  zeros_acc = jnp.zeros((n, d), jnp.float32)

  g, w = _sc_prep(src, dst, typ, zeros_hist, n=n, r=r)

  wbig = jnp.concatenate(
      [jnp.transpose(rel_W, (0, 2, 1, 3)).reshape(nl, d, r * d),
       root_W], axis=2)

  t = _tc_first(x, W_in, b_in.reshape(1, d), wbig[0])
  h = None
  for l in range(nl):
    p = _sc_scatter(t.reshape(n * (r + 1), d), g, dst, w, zeros_acc, n=n, d=d)
    if l < nl - 1:
      t = _tc_mid(p, t, root_b[l].reshape(1, d), wbig[l + 1], r)
    else:
      h = _tc_last(p, t, root_b[l].reshape(1, d), r)
  return (h, edge_distance)


# speedup vs baseline: 13.0135x; 13.0135x over previous
import functools

import jax
import jax.numpy as jnp
from jax import lax
from jax.experimental import pallas as pl
from jax.experimental.pallas import tpu as pltpu
from jax.experimental.pallas import tpu_sc as plsc

NC = 2
NS = 16
NW = NC * NS
LANES = 16
K = 80


def _mesh():
  return plsc.VectorSubcoreMesh(core_axis_name="c", subcore_axis_name="s",
                                num_cores=NC)


@functools.partial(jax.jit, static_argnames=("n", "r"))
def _sc_prep(src, dst, typ, zeros_hist, *, n, r):
  e = src.shape[0]
  nr = n * r
  per_tile_a = e // NS
  per_tile_b = e // NW
  nb_a = per_tile_a // K
  nb_b = per_tile_b // K
  stripe = nr // NS

  @functools.partial(
      pl.kernel,
      out_type=(jax.ShapeDtypeStruct((e,), jnp.int32),
                jax.ShapeDtypeStruct((e,), jnp.float32)),
      mesh=_mesh(),
      scratch_types=[
          pltpu.VMEM((K,), jnp.int32),
          pltpu.VMEM((K,), jnp.int32),
          pltpu.VMEM((K,), jnp.int32),
          pltpu.VMEM((K,), jnp.int32),
          pltpu.VMEM((K,), jnp.float32),
          pltpu.VMEM((K,), jnp.int32),
          pltpu.VMEM((K,), jnp.float32),
          pltpu.VMEM((nr,), jnp.float32),
          pltpu.VMEM_SHARED((nr,), jnp.float32),
      ],
      compiler_params=pltpu.CompilerParams(needs_layout_passes=False),
  )
  def prep(src_h, dst_h, typ_h, zeros_h, g_h, w_h,
           src_v, dst_v, typ_v, key_v, ones_v, g_v, w_v, hist_v, hist_sh):
    sid = lax.axis_index("s")
    cid = lax.axis_index("c")
    wid = sid * NC + cid

    pltpu.sync_copy(zeros_h, hist_v.at[pl.ds(0, stripe)])
    pltpu.sync_copy(hist_v.at[pl.ds(0, stripe)],
                    hist_sh.at[pl.ds(sid * stripe, stripe)])
    for j in range(K // LANES):
      ones_v[pl.ds(j * LANES, LANES)] = jnp.full((LANES,), 1.0, jnp.float32)
    plsc.subcore_barrier()

    base_a = sid * per_tile_a

    def body_a(b, carry):
      off = base_a + b * K
      pltpu.sync_copy(dst_h.at[pl.ds(off, K)], dst_v)
      pltpu.sync_copy(typ_h.at[pl.ds(off, K)], typ_v)
      for j in range(K // LANES):
        sl = pl.ds(j * LANES, LANES)
        key_v[sl] = dst_v[sl] * r + typ_v[sl]
      pltpu.sync_copy(ones_v, hist_sh.at[key_v], add=True)
      return carry

    lax.fori_loop(0, nb_a, body_a, 0)
    plsc.subcore_barrier()

    pltpu.sync_copy(hist_sh, hist_v)
    base_b = wid * per_tile_b

    def body_b(b, carry):
      off = base_b + b * K
      pltpu.sync_copy(src_h.at[pl.ds(off, K)], src_v)
      pltpu.sync_copy(dst_h.at[pl.ds(off, K)], dst_v)
      pltpu.sync_copy(typ_h.at[pl.ds(off, K)], typ_v)
      for j in range(K // LANES):
        sl = pl.ds(j * LANES, LANES)
        t = typ_v[sl]
        g_v[sl] = src_v[sl] * (r + 1) + t
        cnt = plsc.load_gather(hist_v, [dst_v[sl] * r + t])
        w_v[sl] = 1.0 / jnp.maximum(cnt, 1.0)
      pltpu.sync_copy(g_v, g_h.at[pl.ds(off, K)])
      pltpu.sync_copy(w_v, w_h.at[pl.ds(off, K)])
      return carry

    lax.fori_loop(0, nb_b, body_b, 0)

  return prep(src, dst, typ, zeros_hist)


@functools.partial(jax.jit, static_argnames=("n", "d"))
def _sc_scatter(table, g, dst, w, zeros_acc, *, n, d):
  e = g.shape[0]
  per_tile = e // NW
  nb = per_tile // K
  total_chunks = n // K
  NZC = -(-total_chunks // NS)
  NZC_LAST = total_chunks - (NS - 1) * NZC

  @functools.partial(
      pl.kernel,
      out_type=jax.ShapeDtypeStruct((NC, n, d), jnp.float32),
      mesh=_mesh(),
      scratch_types=[
          pltpu.VMEM((K,), jnp.int32),
          pltpu.VMEM((K,), jnp.int32),
          pltpu.VMEM((K,), jnp.float32),
          pltpu.VMEM((K, d), jnp.float32),
          pltpu.VMEM_SHARED((n, d), jnp.float32),
          pltpu.SemaphoreType.DMA,
      ],
  )
  def scat(t_h, g_h, dst_h, w_h, zeros_h, out_h,
           g_v, dst_v, w_v, rows_v, acc_sh, sem):
    sid = lax.axis_index("s")
    cid = lax.axis_index("c")
    wid = sid * NC + cid

    pltpu.sync_copy(zeros_h, rows_v)

    def zinit(c, carry):
      pltpu.sync_copy(rows_v, acc_sh.at[pl.ds(sid * (K * NZC) + c * K, K)])
      return carry

    nz = jnp.where(sid == NS - 1, NZC_LAST, NZC)
    lax.fori_loop(0, nz, zinit, 0)
    plsc.subcore_barrier()

    base = wid * per_tile

    def body(b, carry):
      off = base + b * K
      pltpu.sync_copy(g_h.at[pl.ds(off, K)], g_v)
      pltpu.sync_copy(dst_h.at[pl.ds(off, K)], dst_v)
      pltpu.sync_copy(w_h.at[pl.ds(off, K)], w_v)
      pltpu.async_copy(t_h.at[g_v], rows_v, sem).wait()

      def scale(j, c2):
        base_i = j * LANES
        wv = w_v[pl.ds(base_i, LANES)]
        for ii in range(LANES):
          ws = wv[ii]
          for k in range(d // LANES):
            sl = pl.ds(k * LANES, LANES)
            rows_v[base_i + ii, sl] = rows_v[base_i + ii, sl] * ws
        return c2

      lax.fori_loop(0, K // LANES, scale, 0)
      pltpu.sync_copy(rows_v, acc_sh.at[dst_v], add=True)
      return carry

    lax.fori_loop(0, nb, body, 0)
    plsc.subcore_barrier()

    def flush(c, carry):
      row0 = sid * (K * NZC) + c * K
      pltpu.sync_copy(acc_sh.at[pl.ds(row0, K)], rows_v)
      pltpu.sync_copy(rows_v, out_h.at[cid, pl.ds(row0, K)])
      return carry

    lax.fori_loop(0, nz, flush, 0)

  return scat(table, g, dst, w, zeros_acc)


_BN = 1000


def _tc_first_body(x_ref, w_ref, b_ref, wb_ref, o_ref):
  h = jnp.dot(x_ref[...], w_ref[...], preferred_element_type=jnp.float32)
  h = jnp.maximum(h + b_ref[...], 0.0)
  o_ref[...] = jnp.dot(h, wb_ref[...], preferred_element_type=jnp.float32)


def _tc_first(x, w_in, b_in, wbig):
  n, d = x.shape
  do = wbig.shape[1]
  return pl.pallas_call(
      _tc_first_body,
      grid=(n // _BN,),
      in_specs=[
          pl.BlockSpec((_BN, d), lambda i: (i, 0)),
          pl.BlockSpec((d, d), lambda i: (0, 0)),
          pl.BlockSpec((1, d), lambda i: (0, 0)),
          pl.BlockSpec((d, do), lambda i: (0, 0)),
      ],
      out_specs=pl.BlockSpec((_BN, do), lambda i: (i, 0)),
      out_shape=jax.ShapeDtypeStruct((n, do), jnp.float32),
  )(x, w_in, b_in, wbig)


def _tc_mid_body(p_ref, root_ref, b_ref, wb_ref, o_ref):
  p = p_ref[...]
  h = jnp.maximum(p[0] + p[1] + root_ref[...] + b_ref[...], 0.0)
  o_ref[...] = jnp.dot(h, wb_ref[...], preferred_element_type=jnp.float32)


def _tc_mid(p, t_prev, b, wbig, r):
  n, d = p.shape[1], p.shape[2]
  do = wbig.shape[1]
  return pl.pallas_call(
      _tc_mid_body,
      grid=(n // _BN,),
      in_specs=[
          pl.BlockSpec((2, _BN, d), lambda i: (0, i, 0)),
          pl.BlockSpec((_BN, d), lambda i: (i, r)),
          pl.BlockSpec((1, d), lambda i: (0, 0)),
          pl.BlockSpec((d, do), lambda i: (0, 0)),
      ],
      out_specs=pl.BlockSpec((_BN, do), lambda i: (i, 0)),
      out_shape=jax.ShapeDtypeStruct((n, do), jnp.float32),
  )(p, t_prev, b, wbig)


def _tc_last_body(p_ref, root_ref, b_ref, o_ref):
  p = p_ref[...]
  o_ref[...] = jnp.maximum(p[0] + p[1] + root_ref[...] + b_ref[...], 0.0)


def _tc_last(p, t_prev, b, r):
  n, d = p.shape[1], p.shape[2]
  return pl.pallas_call(
      _tc_last_body,
      grid=(n // _BN,),
      in_specs=[
          pl.BlockSpec((2, _BN, d), lambda i: (0, i, 0)),
          pl.BlockSpec((_BN, d), lambda i: (i, r)),
          pl.BlockSpec((1, d), lambda i: (0, 0)),
      ],
      out_specs=pl.BlockSpec((_BN, d), lambda i: (i, 0)),
      out_shape=jax.ShapeDtypeStruct((n, d), jnp.float32),
  )(p, t_prev, b)


def kernel(x, edge_index, edge_attr, W_in, b_in, rel_W, root_W, root_b):
  n, d = x.shape
  e = edge_index.shape[1]
  nl, r = rel_W.shape[0], rel_W.shape[1]

  src = edge_index[0]
  dst = edge_index[1]
  typ = edge_attr[:, 1].astype(jnp.int32)
  edge_distance = edge_attr[:, 0].astype(jnp.float32)

  zeros_hist = jnp.zeros((n * r // NS,), jnp.float32)
  zeros_acc = jnp.zeros((K, d), jnp.float32)

  g, w = _sc_prep(src, dst, typ, zeros_hist, n=n, r=r)

  wbig = jnp.concatenate(
      [jnp.transpose(rel_W, (0, 2, 1, 3)).reshape(nl, d, r * d),
       root_W], axis=2)

  t = _tc_first(x, W_in, b_in.reshape(1, d), wbig[0])
  h = None
  for l in range(nl):
    p = _sc_scatter(t.reshape(n * (r + 1), d), g, dst, w, zeros_acc, n=n, d=d)
    if l < nl - 1:
      t = _tc_mid(p, t, root_b[l].reshape(1, d), wbig[l + 1], r)
    else:
      h = _tc_last(p, t, root_b[l].reshape(1, d), r)
  return (h, edge_distance)


# speedup vs baseline: 24.1347x; 1.8546x over previous
import functools

import jax
import jax.numpy as jnp
from jax import lax
from jax.experimental import pallas as pl
from jax.experimental.pallas import tpu as pltpu
from jax.experimental.pallas import tpu_sc as plsc

NC = 2
NS = 16
NW = NC * NS
LANES = 16
K = 80


def _mesh():
  return plsc.VectorSubcoreMesh(core_axis_name="c", subcore_axis_name="s",
                                num_cores=NC)


@functools.partial(jax.jit, static_argnames=("n", "r"))
def _sc_prep(src, dst, typ, zeros_hist, *, n, r):
  e = src.shape[0]
  nr = n * r
  per_tile_a = e // NS
  per_tile_b = e // NW
  nb_a = per_tile_a // K
  nb_b = per_tile_b // K
  stripe = nr // NS
  lanes_b = per_tile_b // LANES

  @functools.partial(
      pl.kernel,
      out_type=(jax.ShapeDtypeStruct((e,), jnp.int32),
                jax.ShapeDtypeStruct((e,), jnp.float32)),
      mesh=_mesh(),
      scratch_types=[
          pltpu.VMEM((4 * per_tile_b,), jnp.int32),
          pltpu.VMEM((nb_a, K), jnp.int32),
          pltpu.VMEM((per_tile_b,), jnp.float32),
          pltpu.VMEM((per_tile_b,), jnp.float32),
          pltpu.VMEM((K,), jnp.float32),
          pltpu.VMEM_SHARED((nr,), jnp.float32),
          pltpu.SemaphoreType.DMA,
      ],
  )
  def prep(src_h, dst_h, typ_h, zeros_h, g_h, w_h,
           big_v, key_v, cnt_v, w_v, ones_v, hist_sh, sem):
    sid = lax.axis_index("s")
    cid = lax.axis_index("c")
    wid = sid * NC + cid
    pb = per_tile_b

    pltpu.sync_copy(zeros_h, cnt_v.at[pl.ds(0, stripe)])
    pltpu.sync_copy(cnt_v.at[pl.ds(0, stripe)],
                    hist_sh.at[pl.ds(sid * stripe, stripe)])
    for j in range(K // LANES):
      ones_v[pl.ds(j * LANES, LANES)] = jnp.full((LANES,), 1.0, jnp.float32)

    base_a = sid * per_tile_a
    pltpu.sync_copy(dst_h.at[pl.ds(base_a, per_tile_a)],
                    big_v.at[pl.ds(0, per_tile_a)])
    pltpu.sync_copy(typ_h.at[pl.ds(base_a, per_tile_a)],
                    big_v.at[pl.ds(per_tile_a, per_tile_a)])
    plsc.subcore_barrier()

    def body_a(b, carry):
      for j in range(K // LANES):
        o = b * K + j * LANES
        d16 = big_v[pl.ds(o, LANES)]
        t16 = big_v[pl.ds(per_tile_a + o, LANES)]
        key_v[b, pl.ds(j * LANES, LANES)] = d16 * r + t16
      pltpu.async_copy(ones_v, hist_sh.at[key_v.at[b]], sem, add=True)
      return carry

    lax.fori_loop(0, nb_a, body_a, 0)
    pltpu.make_async_copy(dst_h.at[pl.ds(base_a, per_tile_a)],
                          big_v.at[pl.ds(0, per_tile_a)], sem).wait()
    plsc.subcore_barrier()

    base_b = wid * pb
    pltpu.sync_copy(src_h.at[pl.ds(base_b, pb)], big_v.at[pl.ds(0, pb)])
    pltpu.sync_copy(dst_h.at[pl.ds(base_b, pb)], big_v.at[pl.ds(pb, pb)])
    pltpu.sync_copy(typ_h.at[pl.ds(base_b, pb)], big_v.at[pl.ds(2 * pb, pb)])

    def body_b(b, carry):
      for j in range(K // LANES):
        o = b * K + j * LANES
        s16 = big_v[pl.ds(o, LANES)]
        d16 = big_v[pl.ds(pb + o, LANES)]
        t16 = big_v[pl.ds(2 * pb + o, LANES)]
        big_v[pl.ds(3 * pb + o, LANES)] = s16 * (r + 1) + t16
        key_v[b, pl.ds(j * LANES, LANES)] = d16 * r + t16
      pltpu.async_copy(hist_sh.at[key_v.at[b]], cnt_v.at[pl.ds(b * K, K)], sem)
      return carry

    lax.fori_loop(0, nb_b, body_b, 0)
    pltpu.make_async_copy(src_h.at[pl.ds(base_b, pb)],
                          cnt_v, sem).wait()

    def body_w(i, carry):
      sl = pl.ds(i * LANES, LANES)
      w_v[sl] = 1.0 / jnp.maximum(cnt_v[sl], 1.0)
      return carry

    lax.fori_loop(0, lanes_b, body_w, 0)
    pltpu.sync_copy(big_v.at[pl.ds(3 * pb, pb)], g_h.at[pl.ds(base_b, pb)])
    pltpu.sync_copy(w_v, w_h.at[pl.ds(base_b, pb)])

  return prep(src, dst, typ, zeros_hist)


NBUF = 2
SCH = 42


@functools.partial(jax.jit, static_argnames=("n", "d"))
def _sc_scatter(table, g4, dst4, w4, zeros_acc, *, n, d):
  nsc = g4.shape[0] // (NW * SCH * K)
  total_chunks = n // K
  NZC = -(-total_chunks // NS)
  NZC_LAST = total_chunks - (NS - 1) * NZC

  @functools.partial(
      pl.kernel,
      out_type=jax.ShapeDtypeStruct((NC, n, d), jnp.float32),
      mesh=_mesh(),
      scratch_types=[
          pltpu.VMEM((NBUF * SCH * K,), jnp.int32),
          pltpu.VMEM((NBUF * SCH * K,), jnp.int32),
          pltpu.VMEM((NBUF * SCH * K,), jnp.float32),
          pltpu.VMEM((K,), jnp.int32),
          pltpu.VMEM((K,), jnp.int32),
          pltpu.VMEM((NBUF, K, d), jnp.float32),
          pltpu.VMEM_SHARED((n, d), jnp.float32),
          pltpu.SemaphoreType.DMA((NBUF,)),
          pltpu.SemaphoreType.DMA((NBUF,)),
          pltpu.SemaphoreType.DMA((NBUF,)),
      ],
  )
  def scat(t_h, g_h, dst_h, w_h, zeros_h, out_h,
           g_v, dst_v, w_v, di0_v, di1_v, rows_v, acc_sh, gsem, ssem, msem):
    di_v = (di0_v, di1_v)
    ck = SCH * K
    sid = lax.axis_index("s")
    cid = lax.axis_index("c")
    wid = sid * NC + cid

    hbase = wid * (nsc * ck)

    pltpu.async_copy(g_h.at[pl.ds(hbase, ck)], g_v.at[pl.ds(0, ck)],
                     msem.at[0])
    pltpu.async_copy(dst_h.at[pl.ds(hbase, ck)], dst_v.at[pl.ds(0, ck)],
                     msem.at[0])
    pltpu.async_copy(w_h.at[pl.ds(hbase, ck)], w_v.at[pl.ds(0, ck)],
                     msem.at[0])

    pltpu.sync_copy(zeros_h, rows_v.at[0])

    def zinit(c, carry):
      pltpu.sync_copy(rows_v.at[0],
                      acc_sh.at[pl.ds(sid * (K * NZC) + c * K, K)])
      return carry

    nz = jnp.where(sid == NS - 1, NZC_LAST, NZC)
    lax.fori_loop(0, nz, zinit, 0)
    plsc.subcore_barrier()

    for s in range(nsc):
      m = s % NBUF
      if s > 0:
        pltpu.make_async_copy(rows_v.at[1], acc_sh.at[di_v[1]],
                              ssem.at[1]).wait()
      if s + 1 < nsc:
        mn = (s + 1) % NBUF
        hoff = hbase + (s + 1) * ck
        pltpu.async_copy(g_h.at[pl.ds(hoff, ck)],
                         g_v.at[pl.ds(mn * ck, ck)], msem.at[mn])
        pltpu.async_copy(dst_h.at[pl.ds(hoff, ck)],
                         dst_v.at[pl.ds(mn * ck, ck)], msem.at[mn])
        pltpu.async_copy(w_h.at[pl.ds(hoff, ck)],
                         w_v.at[pl.ds(mn * ck, ck)], msem.at[mn])
      hcur = hbase + s * ck
      pltpu.make_async_copy(g_h.at[pl.ds(hcur, ck)],
                            g_v.at[pl.ds(m * ck, ck)], msem.at[m]).wait()
      pltpu.make_async_copy(dst_h.at[pl.ds(hcur, ck)],
                            dst_v.at[pl.ds(m * ck, ck)], msem.at[m]).wait()
      pltpu.make_async_copy(w_h.at[pl.ds(hcur, ck)],
                            w_v.at[pl.ds(m * ck, ck)], msem.at[m]).wait()
      pltpu.async_copy(t_h.at[g_v.at[pl.ds(m * ck, K)]], rows_v.at[0],
                       gsem.at[0])

      def inner(gi, carry, m=m):
        for j in range(NBUF):
          i = gi * NBUF + j
          jn = (j + 1) % NBUF

          @pl.when(i >= 1)
          def _():
            pltpu.make_async_copy(rows_v.at[jn], acc_sh.at[di_v[jn]],
                                  ssem.at[jn]).wait()

          @pl.when(i + 1 < SCH)
          def _():
            pltpu.async_copy(t_h.at[g_v.at[pl.ds(m * ck + (i + 1) * K, K)]],
                             rows_v.at[jn], gsem.at[jn])

          pltpu.make_async_copy(t_h.at[g_v.at[pl.ds(m * ck + i * K, K)]],
                                rows_v.at[j], gsem.at[j]).wait()

          for grp in range(K // LANES):
            di_v[j][pl.ds(grp * LANES, LANES)] = (
                dst_v[pl.ds(m * ck + i * K + grp * LANES, LANES)])

          def scale(grp, c2):
            base_i = grp * LANES
            wv = w_v[pl.ds(m * ck + i * K + base_i, LANES)]
            for ii in range(LANES):
              ws = wv[ii]
              for k in range(d // LANES):
                sl = pl.ds(k * LANES, LANES)
                rows_v[j, base_i + ii, sl] = rows_v[j, base_i + ii, sl] * ws
            return c2

          lax.fori_loop(0, K // LANES, scale, 0)
          pltpu.async_copy(rows_v.at[j], acc_sh.at[di_v[j]],
                           ssem.at[j], add=True)
        return carry

      lax.fori_loop(0, SCH // NBUF, inner, 0)

    pltpu.make_async_copy(rows_v.at[1], acc_sh.at[di_v[1]],
                          ssem.at[1]).wait()
    plsc.subcore_barrier()

    def flush(c, carry):
      row0 = sid * (K * NZC) + c * K
      pltpu.sync_copy(acc_sh.at[pl.ds(row0, K)], rows_v.at[0])
      pltpu.sync_copy(rows_v.at[0], out_h.at[cid, pl.ds(row0, K)])
      return carry

    lax.fori_loop(0, nz, flush, 0)

  return scat(table, g4, dst4, w4, zeros_acc)


_BN = 1000


def _tc_first_body(x_ref, w_ref, b_ref, wb_ref, o_ref):
  h = jnp.dot(x_ref[...], w_ref[...], preferred_element_type=jnp.float32)
  h = jnp.maximum(h + b_ref[...], 0.0)
  o_ref[...] = jnp.dot(h, wb_ref[...], preferred_element_type=jnp.float32)


def _tc_first(x, w_in, b_in, wbig):
  n, d = x.shape
  do = wbig.shape[1]
  return pl.pallas_call(
      _tc_first_body,
      grid=(n // _BN,),
      in_specs=[
          pl.BlockSpec((_BN, d), lambda i: (i, 0)),
          pl.BlockSpec((d, d), lambda i: (0, 0)),
          pl.BlockSpec((1, d), lambda i: (0, 0)),
          pl.BlockSpec((d, do), lambda i: (0, 0)),
      ],
      out_specs=pl.BlockSpec((_BN, do), lambda i: (i, 0)),
      out_shape=jax.ShapeDtypeStruct((n, do), jnp.float32),
  )(x, w_in, b_in, wbig)


def _tc_mid_body(p_ref, root_ref, b_ref, wb_ref, o_ref):
  p = p_ref[...]
  h = jnp.maximum(p[0] + p[1] + root_ref[...] + b_ref[...], 0.0)
  o_ref[...] = jnp.dot(h, wb_ref[...], preferred_element_type=jnp.float32)


def _tc_mid(p, t_prev, b, wbig, r):
  n, d = p.shape[1], p.shape[2]
  do = wbig.shape[1]
  return pl.pallas_call(
      _tc_mid_body,
      grid=(n // _BN,),
      in_specs=[
          pl.BlockSpec((2, _BN, d), lambda i: (0, i, 0)),
          pl.BlockSpec((_BN, d), lambda i: (i, r)),
          pl.BlockSpec((1, d), lambda i: (0, 0)),
          pl.BlockSpec((d, do), lambda i: (0, 0)),
      ],
      out_specs=pl.BlockSpec((_BN, do), lambda i: (i, 0)),
      out_shape=jax.ShapeDtypeStruct((n, do), jnp.float32),
  )(p, t_prev, b, wbig)


def _tc_last_body(p_ref, root_ref, b_ref, o_ref):
  p = p_ref[...]
  o_ref[...] = jnp.maximum(p[0] + p[1] + root_ref[...] + b_ref[...], 0.0)


def _tc_last(p, t_prev, b, r):
  n, d = p.shape[1], p.shape[2]
  return pl.pallas_call(
      _tc_last_body,
      grid=(n // _BN,),
      in_specs=[
          pl.BlockSpec((2, _BN, d), lambda i: (0, i, 0)),
          pl.BlockSpec((_BN, d), lambda i: (i, r)),
          pl.BlockSpec((1, d), lambda i: (0, 0)),
      ],
      out_specs=pl.BlockSpec((_BN, d), lambda i: (i, 0)),
      out_shape=jax.ShapeDtypeStruct((n, d), jnp.float32),
  )(p, t_prev, b)


def kernel(x, edge_index, edge_attr, W_in, b_in, rel_W, root_W, root_b):
  n, d = x.shape
  e = edge_index.shape[1]
  nl, r = rel_W.shape[0], rel_W.shape[1]

  src = edge_index[0]
  dst = edge_index[1]
  typ = edge_attr[:, 1].astype(jnp.int32)
  edge_distance = edge_attr[:, 0].astype(jnp.float32)

  zeros_hist = jnp.zeros((n * r // NS,), jnp.float32)
  zeros_acc = jnp.zeros((K, d), jnp.float32)

  g, w = _sc_prep(src, dst, typ, zeros_hist, n=n, r=r)

  nb = e // NW // K
  nsc = -(-nb // SCH)
  pad = nsc * SCH - nb

  def _chunked(a):
    a3 = a.reshape(NW, nb, K)
    if pad:
      a3 = jnp.concatenate(
          [a3, jnp.zeros((NW, pad, K), a.dtype)], axis=1)
    return a3.reshape(-1)

  g4 = _chunked(g)
  dst4 = _chunked(dst)
  w4 = _chunked(w)

  wbig = jnp.concatenate(
      [jnp.transpose(rel_W, (0, 2, 1, 3)).reshape(nl, d, r * d),
       root_W], axis=2)

  t = _tc_first(x, W_in, b_in.reshape(1, d), wbig[0])
  h = None
  for l in range(nl):
    p = _sc_scatter(t.reshape(n * (r + 1), d), g4, dst4, w4, zeros_acc,
                    n=n, d=d)
    if l < nl - 1:
      t = _tc_mid(p, t, root_b[l].reshape(1, d), wbig[l + 1], r)
    else:
      h = _tc_last(p, t, root_b[l].reshape(1, d), r)
  return (h, edge_distance)


# speedup vs baseline: 25.1269x; 1.0411x over previous
import functools

import jax
import jax.numpy as jnp
from jax import lax
from jax.experimental import pallas as pl
from jax.experimental.pallas import tpu as pltpu
from jax.experimental.pallas import tpu_sc as plsc

NC = 2
NS = 16
NW = NC * NS
LANES = 16
K = 80


def _mesh():
  return plsc.VectorSubcoreMesh(core_axis_name="c", subcore_axis_name="s",
                                num_cores=NC)


@functools.partial(jax.jit, static_argnames=("n", "r", "pt"))
def _sc_prep(src, dst, typ, zeros_hist, *, n, r, pt):
  e = src.shape[0]
  nr = n * r
  per_tile_a = e // NS
  per_tile_b = e // NW
  nb_a = per_tile_a // K
  nb_b = per_tile_b // K
  stripe = nr // NS
  lanes_b = per_tile_b // LANES
  padw = pt - per_tile_b

  @functools.partial(
      pl.kernel,
      out_type=(jax.ShapeDtypeStruct((NW * pt,), jnp.int32),
                jax.ShapeDtypeStruct((NW * pt,), jnp.int32),
                jax.ShapeDtypeStruct((NW * pt,), jnp.float32)),
      mesh=_mesh(),
      scratch_types=[
          pltpu.VMEM((4 * per_tile_b,), jnp.int32),
          pltpu.VMEM((nb_a, K), jnp.int32),
          pltpu.VMEM((per_tile_b,), jnp.float32),
          pltpu.VMEM((per_tile_b,), jnp.float32),
          pltpu.VMEM((K,), jnp.float32),
          pltpu.VMEM((K,), jnp.int32),
          pltpu.VMEM((K,), jnp.float32),
          pltpu.VMEM_SHARED((nr,), jnp.float32),
          pltpu.SemaphoreType.DMA,
      ],
  )
  def prep(src_h, dst_h, typ_h, zeros_h, g_h, dstp_h, w_h,
           big_v, key_v, cnt_v, w_v, ones_v, zi_v, zf_v, hist_sh, sem):
    sid = lax.axis_index("s")
    cid = lax.axis_index("c")
    wid = sid * NC + cid
    pb = per_tile_b

    pltpu.sync_copy(zeros_h, cnt_v.at[pl.ds(0, stripe)])
    pltpu.sync_copy(cnt_v.at[pl.ds(0, stripe)],
                    hist_sh.at[pl.ds(sid * stripe, stripe)])
    for j in range(K // LANES):
      sl = pl.ds(j * LANES, LANES)
      ones_v[sl] = jnp.full((LANES,), 1.0, jnp.float32)
      zi_v[sl] = jnp.zeros((LANES,), jnp.int32)
      zf_v[sl] = jnp.zeros((LANES,), jnp.float32)

    base_a = sid * per_tile_a
    pltpu.sync_copy(dst_h.at[pl.ds(base_a, per_tile_a)],
                    big_v.at[pl.ds(0, per_tile_a)])
    pltpu.sync_copy(typ_h.at[pl.ds(base_a, per_tile_a)],
                    big_v.at[pl.ds(per_tile_a, per_tile_a)])
    plsc.subcore_barrier()

    def body_a(b, carry):
      for j in range(K // LANES):
        o = b * K + j * LANES
        d16 = big_v[pl.ds(o, LANES)]
        t16 = big_v[pl.ds(per_tile_a + o, LANES)]
        key_v[b, pl.ds(j * LANES, LANES)] = d16 * r + t16
      pltpu.async_copy(ones_v, hist_sh.at[key_v.at[b]], sem, add=True)
      return carry

    lax.fori_loop(0, nb_a, body_a, 0)
    pltpu.make_async_copy(dst_h.at[pl.ds(base_a, per_tile_a)],
                          big_v.at[pl.ds(0, per_tile_a)], sem).wait()
    plsc.subcore_barrier()

    base_b = wid * pb
    pltpu.sync_copy(src_h.at[pl.ds(base_b, pb)], big_v.at[pl.ds(0, pb)])
    pltpu.sync_copy(dst_h.at[pl.ds(base_b, pb)], big_v.at[pl.ds(pb, pb)])
    pltpu.sync_copy(typ_h.at[pl.ds(base_b, pb)], big_v.at[pl.ds(2 * pb, pb)])

    def body_b(b, carry):
      for j in range(K // LANES):
        o = b * K + j * LANES
        s16 = big_v[pl.ds(o, LANES)]
        d16 = big_v[pl.ds(pb + o, LANES)]
        t16 = big_v[pl.ds(2 * pb + o, LANES)]
        big_v[pl.ds(3 * pb + o, LANES)] = s16 * (r + 1) + t16
        key_v[b, pl.ds(j * LANES, LANES)] = d16 * r + t16
      pltpu.async_copy(hist_sh.at[key_v.at[b]], cnt_v.at[pl.ds(b * K, K)], sem)
      return carry

    lax.fori_loop(0, nb_b, body_b, 0)
    pltpu.make_async_copy(src_h.at[pl.ds(base_b, pb)],
                          cnt_v, sem).wait()

    def body_w(i, carry):
      sl = pl.ds(i * LANES, LANES)
      w_v[sl] = 1.0 / jnp.maximum(cnt_v[sl], 1.0)
      return carry

    lax.fori_loop(0, lanes_b, body_w, 0)
    ob = wid * pt
    pltpu.sync_copy(big_v.at[pl.ds(3 * pb, pb)], g_h.at[pl.ds(ob, pb)])
    pltpu.sync_copy(big_v.at[pl.ds(pb, pb)], dstp_h.at[pl.ds(ob, pb)])
    pltpu.sync_copy(w_v, w_h.at[pl.ds(ob, pb)])
    for q in range(padw // K):
      pltpu.sync_copy(zi_v, g_h.at[pl.ds(ob + pb + q * K, K)])
      pltpu.sync_copy(zi_v, dstp_h.at[pl.ds(ob + pb + q * K, K)])
      pltpu.sync_copy(zf_v, w_h.at[pl.ds(ob + pb + q * K, K)])

  return prep(src, dst, typ, zeros_hist)


NBUF = 2


@functools.partial(jax.jit, static_argnames=("n", "d"))
def _sc_scatter(table, g4, dst4, w4, zeros_acc, *, n, d):
  pt = g4.shape[0] // NW
  nbt = pt // K
  total_chunks = n // K
  NZC = -(-total_chunks // NS)
  NZC_LAST = total_chunks - (NS - 1) * NZC

  @functools.partial(
      pl.kernel,
      out_type=jax.ShapeDtypeStruct((NC, n, d), jnp.float32),
      mesh=_mesh(),
      scratch_types=[
          pltpu.VMEM((pt,), jnp.int32),
          pltpu.VMEM((pt,), jnp.int32),
          pltpu.VMEM((pt,), jnp.float32),
          pltpu.VMEM((K,), jnp.int32),
          pltpu.VMEM((K,), jnp.int32),
          pltpu.VMEM((NBUF, K, d), jnp.float32),
          pltpu.VMEM_SHARED((n, d), jnp.float32),
          pltpu.SemaphoreType.DMA((NBUF,)),
          pltpu.SemaphoreType.DMA((NBUF,)),
          pltpu.SemaphoreType.DMA,
      ],
  )
  def scat(t_h, g_h, dst_h, w_h, zeros_h, out_h,
           g_v, dst_v, w_v, di0_v, di1_v, rows_v, acc_sh, gsem, ssem, msem):
    di_v = (di0_v, di1_v)
    sid = lax.axis_index("s")
    cid = lax.axis_index("c")
    wid = sid * NC + cid
    hbase = wid * pt

    pltpu.async_copy(g_h.at[pl.ds(hbase, pt)], g_v, msem)
    pltpu.async_copy(dst_h.at[pl.ds(hbase, pt)], dst_v, msem)
    pltpu.async_copy(w_h.at[pl.ds(hbase, pt)], w_v, msem)

    pltpu.sync_copy(zeros_h, rows_v.at[0])

    def zinit(c, carry):
      pltpu.sync_copy(rows_v.at[0],
                      acc_sh.at[pl.ds(sid * (K * NZC) + c * K, K)])
      return carry

    nz = jnp.where(sid == NS - 1, NZC_LAST, NZC)
    lax.fori_loop(0, nz, zinit, 0)

    pltpu.make_async_copy(g_h.at[pl.ds(hbase, pt)], g_v, msem).wait()
    pltpu.make_async_copy(dst_h.at[pl.ds(hbase, pt)], dst_v, msem).wait()
    pltpu.make_async_copy(w_h.at[pl.ds(hbase, pt)], w_v, msem).wait()
    pltpu.async_copy(t_h.at[g_v.at[pl.ds(0, K)]], rows_v.at[0], gsem.at[0])
    plsc.subcore_barrier()

    def inner(gi, carry):
      for j in range(NBUF):
        i = gi * NBUF + j
        jn = (j + 1) % NBUF

        for grp in range(K // LANES):
          di_v[j][pl.ds(grp * LANES, LANES)] = (
              dst_v[pl.ds(i * K + grp * LANES, LANES)])

        @pl.when(i >= 1)
        def _():
          pltpu.make_async_copy(rows_v.at[jn], acc_sh.at[di_v[jn]],
                                ssem.at[jn]).wait()

        @pl.when(i + 1 < nbt)
        def _():
          pltpu.async_copy(t_h.at[g_v.at[pl.ds((i + 1) * K, K)]],
                           rows_v.at[jn], gsem.at[jn])

        pltpu.make_async_copy(t_h.at[g_v.at[pl.ds(i * K, K)]],
                              rows_v.at[j], gsem.at[j]).wait()

        def scale(grp, c2):
          base_i = grp * LANES
          wv = w_v[pl.ds(i * K + base_i, LANES)]
          for ii in range(LANES):
            ws = wv[ii]
            for k in range(d // LANES):
              sl = pl.ds(k * LANES, LANES)
              rows_v[j, base_i + ii, sl] = rows_v[j, base_i + ii, sl] * ws
          return c2

        lax.fori_loop(0, K // LANES, scale, 0)
        pltpu.async_copy(rows_v.at[j], acc_sh.at[di_v[j]],
                         ssem.at[j], add=True)
      return carry

    lax.fori_loop(0, nbt // NBUF, inner, 0)
    pltpu.make_async_copy(rows_v.at[1], acc_sh.at[di_v[1]],
                          ssem.at[1]).wait()
    plsc.subcore_barrier()

    def flush(c, carry):
      row0 = sid * (K * NZC) + c * K
      pltpu.sync_copy(acc_sh.at[pl.ds(row0, K)], rows_v.at[0])
      pltpu.sync_copy(rows_v.at[0], out_h.at[cid, pl.ds(row0, K)])
      return carry

    lax.fori_loop(0, nz, flush, 0)

  return scat(table, g4, dst4, w4, zeros_acc)


_BN = 1000


def _mm_out(h, rel_ref, root_ref, o_ref):
  r, d = rel_ref.shape[0], rel_ref.shape[1]
  for rr in range(r):
    o_ref[:, pl.ds(rr * d, d)] = jnp.dot(
        h, rel_ref[rr], preferred_element_type=jnp.float32)
  o_ref[:, pl.ds(r * d, d)] = jnp.dot(
      h, root_ref[...], preferred_element_type=jnp.float32)


def _tc_first_body(x_ref, w_ref, b_ref, rel_ref, root_ref, o_ref):
  h = jnp.dot(x_ref[...], w_ref[...], preferred_element_type=jnp.float32)
  h = jnp.maximum(h + b_ref[...], 0.0)
  _mm_out(h, rel_ref, root_ref, o_ref)


def _tc_first(x, w_in, b_in, rel, root):
  n, d = x.shape
  r = rel.shape[0]
  do = (r + 1) * d
  return pl.pallas_call(
      _tc_first_body,
      grid=(n // _BN,),
      in_specs=[
          pl.BlockSpec((_BN, d), lambda i: (i, 0)),
          pl.BlockSpec((d, d), lambda i: (0, 0)),
          pl.BlockSpec((1, d), lambda i: (0, 0)),
          pl.BlockSpec((r, d, d), lambda i: (0, 0, 0)),
          pl.BlockSpec((d, d), lambda i: (0, 0)),
      ],
      out_specs=pl.BlockSpec((_BN, do), lambda i: (i, 0)),
      out_shape=jax.ShapeDtypeStruct((n, do), jnp.float32),
  )(x, w_in, b_in, rel, root)


def _tc_mid_body(p_ref, root_ref, b_ref, rel_ref, rootw_ref, o_ref):
  p = p_ref[...]
  h = jnp.maximum(p[0] + p[1] + root_ref[...] + b_ref[...], 0.0)
  _mm_out(h, rel_ref, rootw_ref, o_ref)


def _tc_mid(p, t_prev, b, rel, root, r):
  n, d = p.shape[1], p.shape[2]
  do = (r + 1) * d
  return pl.pallas_call(
      _tc_mid_body,
      grid=(n // _BN,),
      in_specs=[
          pl.BlockSpec((2, _BN, d), lambda i: (0, i, 0)),
          pl.BlockSpec((_BN, d), lambda i: (i, r)),
          pl.BlockSpec((1, d), lambda i: (0, 0)),
          pl.BlockSpec((r, d, d), lambda i: (0, 0, 0)),
          pl.BlockSpec((d, d), lambda i: (0, 0)),
      ],
      out_specs=pl.BlockSpec((_BN, do), lambda i: (i, 0)),
      out_shape=jax.ShapeDtypeStruct((n, do), jnp.float32),
  )(p, t_prev, b, rel, root)


def _tc_last_body(p_ref, root_ref, b_ref, o_ref):
  p = p_ref[...]
  o_ref[...] = jnp.maximum(p[0] + p[1] + root_ref[...] + b_ref[...], 0.0)


def _tc_last(p, t_prev, b, r):
  n, d = p.shape[1], p.shape[2]
  return pl.pallas_call(
      _tc_last_body,
      grid=(n // _BN,),
      in_specs=[
          pl.BlockSpec((2, _BN, d), lambda i: (0, i, 0)),
          pl.BlockSpec((_BN, d), lambda i: (i, r)),
          pl.BlockSpec((1, d), lambda i: (0, 0)),
      ],
      out_specs=pl.BlockSpec((_BN, d), lambda i: (i, 0)),
      out_shape=jax.ShapeDtypeStruct((n, d), jnp.float32),
  )(p, t_prev, b)


def kernel(x, edge_index, edge_attr, W_in, b_in, rel_W, root_W, root_b):
  n, d = x.shape
  e = edge_index.shape[1]
  nl, r = rel_W.shape[0], rel_W.shape[1]

  src = edge_index[0]
  dst = edge_index[1]
  typ = edge_attr[:, 1].astype(jnp.int32)
  edge_distance = edge_attr[:, 0].astype(jnp.float32)

  zeros_hist = jnp.zeros((n * r // NS,), jnp.float32)
  zeros_acc = jnp.zeros((K, d), jnp.float32)

  nb = e // NW // K
  nbt = nb + (nb % NBUF)
  pt = nbt * K

  g4, dst4, w4 = _sc_prep(src, dst, typ, zeros_hist, n=n, r=r, pt=pt)

  t = _tc_first(x, W_in, b_in.reshape(1, d), rel_W[0], root_W[0])
  h = None
  for l in range(nl):
    p = _sc_scatter(t.reshape(n * (r + 1), d), g4, dst4, w4, zeros_acc,
                    n=n, d=d)
    if l < nl - 1:
      t = _tc_mid(p, t, root_b[l].reshape(1, d), rel_W[l + 1], root_W[l + 1],
                  r)
    else:
      h = _tc_last(p, t, root_b[l].reshape(1, d), r)
  return (h, edge_distance)
